# baseline (speedup 1.0000x reference)
import functools

import jax
import jax.numpy as jnp
from jax import lax
from jax.experimental import pallas as pl
from jax.experimental.pallas import tpu as pltpu
from jax.experimental.pallas import tpu_sc as plsc

B, L, D = 4, 8192, 768
NC, NS = 2, 16
NW = NC * NS
ROWS_PER_W = L // NW
CH = 64
CHW = CH * D
NCHUNK = ROWS_PER_W // CH

_mesh = plsc.VectorSubcoreMesh(core_axis_name="c", subcore_axis_name="s")


@functools.partial(
    pl.kernel,
    mesh=_mesh,
    out_type=jax.ShapeDtypeStruct((B, L * D), jnp.float32),
    scratch_types=[
        pltpu.VMEM((CHW,), jnp.float32),
        pltpu.VMEM((CHW,), jnp.float32),
    ],
)
def _sc_add(x_hbm, t_hbm, out_hbm, t_buf, x_buf):
    wid = lax.axis_index("s") * NC + lax.axis_index("c")
    base = wid * (ROWS_PER_W * D)

    def chunk_body(k, _):
        off = base + k * CHW
        pltpu.sync_copy(t_hbm.at[pl.ds(off, CHW)], t_buf)

        def batch_body(b, _):
            pltpu.sync_copy(x_hbm.at[b, pl.ds(off, CHW)], x_buf)

            def vec_body(i, _):
                s = pl.ds(i * 16, 16)
                x_buf[s] = x_buf[s] + t_buf[s]
                return 0

            lax.fori_loop(0, CHW // 16, vec_body, 0)
            pltpu.sync_copy(x_buf, out_hbm.at[b, pl.ds(off, CHW)])
            return 0

        lax.fori_loop(0, B, batch_body, 0)
        return 0

    lax.fori_loop(0, NCHUNK, chunk_body, 0)


def kernel(x, table):
    b, l, d = x.shape
    out = _sc_add(x.reshape(b, l * d), table.reshape(l * d))
    return out.reshape(b, l, d)


# speedup vs baseline: 1.8467x; 1.8467x over previous
import functools

import jax
import jax.numpy as jnp
from jax import lax
from jax.experimental import pallas as pl
from jax.experimental.pallas import tpu as pltpu
from jax.experimental.pallas import tpu_sc as plsc

B, L, D = 4, 8192, 768
NC, NS = 2, 16
NW = NC * NS
ROWS_PER_W = L // NW
CH = 32
CHW = CH * D
NCHUNK = ROWS_PER_W // CH
NSTEP = NCHUNK * B
UNROLL = 8

_mesh = plsc.VectorSubcoreMesh(core_axis_name="c", subcore_axis_name="s")


@functools.partial(
    pl.kernel,
    mesh=_mesh,
    out_type=jax.ShapeDtypeStruct((B, L * D), jnp.float32),
    scratch_types=[
        pltpu.VMEM((CHW,), jnp.float32),
        pltpu.VMEM((CHW,), jnp.float32),
        pltpu.VMEM((CHW,), jnp.float32),
        pltpu.VMEM((CHW,), jnp.float32),
        pltpu.VMEM((CHW,), jnp.float32),
        pltpu.SemaphoreType.DMA((3,)),
        pltpu.SemaphoreType.DMA((3,)),
        pltpu.SemaphoreType.DMA((2,)),
    ],
)
def _sc_add(x_hbm, t_hbm, out_hbm, xb0, xb1, xb2, tb0, tb1,
            in_sem, out_sem, t_sem):
    x_bufs = (xb0, xb1, xb2)
    t_bufs = (tb0, tb1)
    wid = lax.axis_index("s") * NC + lax.axis_index("c")
    base = wid * (ROWS_PER_W * D)

    def x_off(step):
        k, b = divmod(step, B)
        return b, base + k * CHW

    t_hand = {0: pltpu.async_copy(
        t_hbm.at[pl.ds(base, CHW)], t_bufs[0], t_sem.at[0])}
    b0, off0 = x_off(0)
    in_hand = {0: pltpu.async_copy(
        x_hbm.at[b0, pl.ds(off0, CHW)], x_bufs[0], in_sem.at[0])}
    out_hand = {}

    for t in range(NSTEP):
        k, b = divmod(t, B)
        p = t % 3
        if b == 0:
            t_hand.pop(k).wait()
            if k + 1 < NCHUNK:
                t_hand[k + 1] = pltpu.async_copy(
                    t_hbm.at[pl.ds(base + (k + 1) * CHW, CHW)],
                    t_bufs[(k + 1) % 2], t_sem.at[(k + 1) % 2])
        in_hand.pop(t).wait()
        if t + 1 < NSTEP:
            q = (t + 1) % 3
            if t - 2 >= 0:
                out_hand.pop(t - 2).wait()
            bn, offn = x_off(t + 1)
            in_hand[t + 1] = pltpu.async_copy(
                x_hbm.at[bn, pl.ds(offn, CHW)], x_bufs[q], in_sem.at[q])

        xb = x_bufs[p]
        tb = t_bufs[k % 2]

        def vec_body(i, _, xb=xb, tb=tb):
            base_w = i * (UNROLL * 16)
            for u in range(UNROLL):
                s = pl.ds(base_w + u * 16, 16)
                plsc.addupdate(xb.at[s], tb[s])
            return 0

        lax.fori_loop(0, CHW // (UNROLL * 16), vec_body, 0)

        _, off = x_off(t)
        out_hand[t] = pltpu.async_copy(
            x_bufs[p], out_hbm.at[b, pl.ds(off, CHW)], out_sem.at[p])

    out_hand.pop(NSTEP - 2).wait()
    out_hand.pop(NSTEP - 1).wait()


def kernel(x, table):
    b, l, d = x.shape
    out = _sc_add(x.reshape(b, l * d), table.reshape(l * d))
    return out.reshape(b, l, d)


# speedup vs baseline: 4.0583x; 2.1976x over previous
import functools

import jax
import jax.numpy as jnp
from jax import lax
from jax.experimental import pallas as pl
from jax.experimental.pallas import tpu as pltpu
from jax.experimental.pallas import tpu_sc as plsc

B, L, D = 4, 8192, 768
NC, NS = 2, 16
NW = NC * NS
ROWS_PER_W = L // NW
CH = 32
NCHUNK = ROWS_PER_W // CH
NSTEP = NCHUNK * B
NVPR = D // 16

_mesh = plsc.VectorSubcoreMesh(core_axis_name="c", subcore_axis_name="s")


@functools.partial(
    pl.kernel,
    mesh=_mesh,
    out_type=jax.ShapeDtypeStruct((B, L, D), jnp.float32),
    scratch_types=[
        pltpu.VMEM((CH, D), jnp.float32),
        pltpu.VMEM((CH, D), jnp.float32),
        pltpu.VMEM((CH, D), jnp.float32),
        pltpu.VMEM((CH, D), jnp.float32),
        pltpu.VMEM((CH, D), jnp.float32),
        pltpu.SemaphoreType.DMA((3,)),
        pltpu.SemaphoreType.DMA((3,)),
        pltpu.SemaphoreType.DMA((2,)),
    ],
)
def _sc_add(x_hbm, t_hbm, out_hbm, xb0, xb1, xb2, tb0, tb1,
            in_sem, out_sem, t_sem):
    x_bufs = (xb0, xb1, xb2)
    t_bufs = (tb0, tb1)
    wid = lax.axis_index("s") * NC + lax.axis_index("c")
    base = wid * ROWS_PER_W

    def x_off(step):
        k, b = divmod(step, B)
        return b, base + k * CH

    t_hand = {0: pltpu.async_copy(
        t_hbm.at[pl.ds(base, CH)], t_bufs[0], t_sem.at[0])}
    b0, row0 = x_off(0)
    in_hand = {0: pltpu.async_copy(
        x_hbm.at[b0, pl.ds(row0, CH)], x_bufs[0], in_sem.at[0])}
    out_hand = {}

    for t in range(NSTEP):
        k, b = divmod(t, B)
        p = t % 3
        if b == 0:
            t_hand.pop(k).wait()
            if k + 1 < NCHUNK:
                t_hand[k + 1] = pltpu.async_copy(
                    t_hbm.at[pl.ds(base + (k + 1) * CH, CH)],
                    t_bufs[(k + 1) % 2], t_sem.at[(k + 1) % 2])
        in_hand.pop(t).wait()
        if t + 1 < NSTEP:
            q = (t + 1) % 3
            if t - 2 >= 0:
                out_hand.pop(t - 2).wait()
            bn, rown = x_off(t + 1)
            in_hand[t + 1] = pltpu.async_copy(
                x_hbm.at[bn, pl.ds(rown, CH)], x_bufs[q], in_sem.at[q])

        xb = x_bufs[p]
        tb = t_bufs[k % 2]

        def vec_body(r, _, xb=xb, tb=tb):
            for c in range(NVPR):
                s = pl.ds(c * 16, 16)
                plsc.addupdate(xb.at[r, s], tb[r, s])
            return 0

        lax.fori_loop(0, CH, vec_body, 0)

        _, row = x_off(t)
        out_hand[t] = pltpu.async_copy(
            x_bufs[p], out_hbm.at[b, pl.ds(row, CH)], out_sem.at[p])

    out_hand.pop(NSTEP - 2).wait()
    out_hand.pop(NSTEP - 1).wait()


def kernel(x, table):
    return _sc_add(x, table)
